# baseline (speedup 1.0000x reference)
import functools

import jax
import jax.numpy as jnp
from jax import lax
from jax.experimental import pallas as pl
from jax.experimental.pallas import tpu as pltpu
from jax.experimental.pallas import tpu_sc as plsc

D_MODEL = 2048
D4 = D_MODEL // 4
N_COMBO = 44
NB = 4
CB = D_MODEL // NB

L = 16
K = 8
NBUF = 4


def _ctable_body(st_ref, at_ref, rt_ref, w_ref, b_ref, out_ref):
    w = w_ref[...]
    ps = jnp.dot(st_ref[...], w[0:D4, :], preferred_element_type=jnp.float32)
    pa = jnp.dot(at_ref[...], w[D4:2 * D4, :], preferred_element_type=jnp.float32)
    pr = jnp.dot(rt_ref[...], w[2 * D4:3 * D4, :], preferred_element_type=jnp.float32)
    bv = b_ref[...]
    for s in range(2):
        for a in range(2):
            out_ref[s * 2 + a, :, :] = pr + (ps[s] + pa[a] + bv)[None, :]


def _build_ctable(subst_table, adj_table, rel_table, W, b):
    c4 = pl.pallas_call(
        _ctable_body,
        grid=(NB,),
        in_specs=[
            pl.BlockSpec((2, D4), lambda c: (0, 0)),
            pl.BlockSpec((2, D4), lambda c: (0, 0)),
            pl.BlockSpec((11, D4), lambda c: (0, 0)),
            pl.BlockSpec((3 * D4, CB), lambda c: (0, c)),
            pl.BlockSpec((CB,), lambda c: (c,)),
        ],
        out_specs=pl.BlockSpec((4, 11, CB), lambda c: (0, 0, c)),
        out_shape=jax.ShapeDtypeStruct((4, 11, D_MODEL), jnp.float32),
    )(subst_table, adj_table, rel_table, W, b)
    return c4.reshape(N_COMBO, D_MODEL)


def _make_sc_lookup(ntok):
    info = plsc.get_sparse_core_info()
    nw = info.num_cores * info.num_subcores
    tpw = ntok // nw
    nch = tpw // K
    mesh = plsc.VectorSubcoreMesh(core_axis_name="c", subcore_axis_name="s")

    ngrp = tpw // K
    gelems = K * D_MODEL
    celems = N_COMBO * D_MODEL
    un = 32

    @functools.partial(
        pl.kernel,
        out_type=jax.ShapeDtypeStruct((ntok * D_MODEL,), jnp.float32),
        mesh=mesh,
        compiler_params=pltpu.CompilerParams(needs_layout_passes=False),
        scratch_types=[
            pltpu.VMEM((tpw,), jnp.int32),
            pltpu.VMEM((tpw,), jnp.int32),
            pltpu.VMEM((tpw,), jnp.int32),
            pltpu.VMEM((celems,), jnp.float32),
            pltpu.VMEM((gelems,), jnp.float32),
            pltpu.VMEM((gelems,), jnp.float32),
            pltpu.SemaphoreType.DMA,
            pltpu.SemaphoreType.DMA,
            pltpu.SemaphoreType.DMA,
        ],
    )
    def sc_lookup(sub_hbm, adj_hbm, rel_hbm, c_hbm, out_hbm,
                  sv, av, rv, cloc, stag0, stag1, semc, sw0, sw1):
        stag = (stag0, stag1)
        semw = (sw0, sw1)
        wid = lax.axis_index("s") * info.num_cores + lax.axis_index("c")
        base = wid * tpw
        pltpu.async_copy(c_hbm, cloc, semc)
        pltpu.sync_copy(sub_hbm.at[pl.ds(base, tpw)], sv)
        pltpu.sync_copy(adj_hbm.at[pl.ds(base, tpw)], av)
        pltpu.sync_copy(rel_hbm.at[pl.ds(base, tpw)], rv)

        for i in range(tpw // L):
            s = sv[pl.ds(i * L, L)]
            a = av[pl.ds(i * L, L)]
            r = rv[pl.ds(i * L, L)]
            c = s * 22 + a * 11 + jnp.clip(r + 5, 0, 10)
            sv[pl.ds(i * L, L)] = jnp.clip(c, 0, N_COMBO - 1)

        pltpu.make_async_copy(c_hbm, cloc, semc).wait()

        lane = lax.iota(jnp.int32, L)

        def wait_write(g, b):
            pltpu.make_async_copy(
                stag[b],
                out_hbm.at[pl.ds((base + g * K) * D_MODEL, gelems)],
                semw[b]).wait()

        def build_group(g, b):
            sb = stag[b]
            for r in range(K):
                perm = lax.bitwise_and(lane + r, K - 1)
                ci = plsc.load_gather(sv, [perm + g * K])
                src0 = ci * D_MODEL + lane
                dst0 = perm * D_MODEL + lane

                def col_step(wi, carry, src0=src0, dst0=dst0):
                    vals = [plsc.load_gather(cloc,
                                             [src0 + (wi * un + j) * L])
                            for j in range(un)]
                    for j in range(un):
                        plsc.store_scatter(sb, [dst0 + (wi * un + j) * L],
                                           vals[j])
                    return carry

                lax.fori_loop(0, D_MODEL // (L * un), col_step, 0)
            pltpu.async_copy(
                stag[b],
                out_hbm.at[pl.ds((base + g * K) * D_MODEL, gelems)],
                semw[b])

        build_group(0, 0)
        build_group(1, 1)

        def outer(go, carry):
            for bb in range(2):
                g = 2 * go + bb
                wait_write(g - 2, bb)
                build_group(g, bb)
            return carry

        lax.fori_loop(1, ngrp // 2, outer, 0)

        wait_write(ngrp - 2, 0)
        wait_write(ngrp - 1, 1)

    return sc_lookup


def kernel(subst_nodes, adj_nodes, rel_positions, subst_table, adj_table,
           rel_table, W, b):
    bdim, sdim = subst_nodes.shape
    ntok = bdim * sdim
    ctable = _build_ctable(subst_table, adj_table, rel_table, W, b)
    sc_lookup = _make_sc_lookup(ntok)
    out = sc_lookup(
        subst_nodes.reshape(ntok).astype(jnp.int32),
        adj_nodes.reshape(ntok).astype(jnp.int32),
        rel_positions.reshape(ntok).astype(jnp.int32),
        ctable.reshape(-1),
    )
    return out.reshape(bdim, sdim, D_MODEL)


# speedup vs baseline: 1.1868x; 1.1868x over previous
import functools

import jax
import jax.numpy as jnp
from jax import lax
from jax.experimental import pallas as pl
from jax.experimental.pallas import tpu as pltpu
from jax.experimental.pallas import tpu_sc as plsc

D_MODEL = 2048
D4 = D_MODEL // 4
N_COMBO = 44
NB = 4
CB = D_MODEL // NB

L = 16
K = 8
NBUF = 4


def _ctable_body(st_ref, at_ref, rt_ref, w_ref, b_ref, out_ref):
    w = w_ref[...]
    ps = jnp.dot(st_ref[...], w[0:D4, :], preferred_element_type=jnp.float32)
    pa = jnp.dot(at_ref[...], w[D4:2 * D4, :], preferred_element_type=jnp.float32)
    pr = jnp.dot(rt_ref[...], w[2 * D4:3 * D4, :], preferred_element_type=jnp.float32)
    bv = b_ref[...]
    for s in range(2):
        for a in range(2):
            out_ref[s * 2 + a, :, :] = pr + (ps[s] + pa[a] + bv)[None, :]


def _build_ctable(subst_table, adj_table, rel_table, W, b):
    c4 = pl.pallas_call(
        _ctable_body,
        grid=(NB,),
        in_specs=[
            pl.BlockSpec((2, D4), lambda c: (0, 0)),
            pl.BlockSpec((2, D4), lambda c: (0, 0)),
            pl.BlockSpec((11, D4), lambda c: (0, 0)),
            pl.BlockSpec((3 * D4, CB), lambda c: (0, c)),
            pl.BlockSpec((CB,), lambda c: (c,)),
        ],
        out_specs=pl.BlockSpec((4, 11, CB), lambda c: (0, 0, c)),
        out_shape=jax.ShapeDtypeStruct((4, 11, D_MODEL), jnp.float32),
    )(subst_table, adj_table, rel_table, W, b)
    return c4.reshape(N_COMBO, D_MODEL)


def _make_sc_lookup(ntok):
    info = plsc.get_sparse_core_info()
    nw = info.num_cores * info.num_subcores
    tpw = ntok // nw
    nch = tpw // K
    mesh = plsc.VectorSubcoreMesh(core_axis_name="c", subcore_axis_name="s")

    ngrp = tpw // K
    gelems = K * D_MODEL
    celems = N_COMBO * D_MODEL
    un = 8

    @functools.partial(
        pl.kernel,
        out_type=jax.ShapeDtypeStruct((ntok * D_MODEL,), jnp.float32),
        mesh=mesh,
        compiler_params=pltpu.CompilerParams(needs_layout_passes=False),
        scratch_types=[
            pltpu.VMEM((tpw,), jnp.int32),
            pltpu.VMEM((tpw,), jnp.int32),
            pltpu.VMEM((tpw,), jnp.int32),
            pltpu.VMEM((celems,), jnp.float32),
            pltpu.VMEM((gelems,), jnp.float32),
            pltpu.VMEM((gelems,), jnp.float32),
            pltpu.SemaphoreType.DMA,
            pltpu.SemaphoreType.DMA,
            pltpu.SemaphoreType.DMA,
        ],
    )
    def sc_lookup(sub_hbm, adj_hbm, rel_hbm, c_hbm, out_hbm,
                  sv, av, rv, cloc, stag0, stag1, semc, sw0, sw1):
        stag = (stag0, stag1)
        semw = (sw0, sw1)
        wid = lax.axis_index("s") * info.num_cores + lax.axis_index("c")
        base = wid * tpw
        pltpu.async_copy(c_hbm, cloc, semc)
        pltpu.sync_copy(sub_hbm.at[pl.ds(base, tpw)], sv)
        pltpu.sync_copy(adj_hbm.at[pl.ds(base, tpw)], av)
        pltpu.sync_copy(rel_hbm.at[pl.ds(base, tpw)], rv)

        for i in range(tpw // L):
            s = sv[pl.ds(i * L, L)]
            a = av[pl.ds(i * L, L)]
            r = rv[pl.ds(i * L, L)]
            c = s * 22 + a * 11 + jnp.clip(r + 5, 0, 10)
            sv[pl.ds(i * L, L)] = jnp.clip(c, 0, N_COMBO - 1)

        pltpu.make_async_copy(c_hbm, cloc, semc).wait()

        lane = lax.iota(jnp.int32, L)

        def wait_write(g, b):
            pltpu.make_async_copy(
                stag[b],
                out_hbm.at[pl.ds((base + g * K) * D_MODEL, gelems)],
                semw[b]).wait()

        def build_group(g, b):
            sb = stag[b]
            for r in range(K):
                perm = lax.bitwise_and(lane + r, K - 1)
                ci = plsc.load_gather(sv, [perm + g * K])
                src0 = ci * D_MODEL + lane
                dst0 = perm * D_MODEL + lane

                def col_step(wi, carry, src0=src0, dst0=dst0):
                    vals = [plsc.load_gather(cloc,
                                             [src0 + (wi * un + j) * L])
                            for j in range(un)]
                    for j in range(un):
                        plsc.store_scatter(sb, [dst0 + (wi * un + j) * L],
                                           vals[j])
                    return carry

                lax.fori_loop(0, D_MODEL // (L * un), col_step, 0)
            pltpu.async_copy(
                stag[b],
                out_hbm.at[pl.ds((base + g * K) * D_MODEL, gelems)],
                semw[b])

        build_group(0, 0)
        build_group(1, 1)

        def outer(go, carry):
            for bb in range(2):
                g = 2 * go + bb
                wait_write(g - 2, bb)
                build_group(g, bb)
            return carry

        lax.fori_loop(1, ngrp // 2, outer, 0)

        wait_write(ngrp - 2, 0)
        wait_write(ngrp - 1, 1)

    return sc_lookup


def kernel(subst_nodes, adj_nodes, rel_positions, subst_table, adj_table,
           rel_table, W, b):
    bdim, sdim = subst_nodes.shape
    ntok = bdim * sdim
    ctable = _build_ctable(subst_table, adj_table, rel_table, W, b)
    sc_lookup = _make_sc_lookup(ntok)
    out = sc_lookup(
        subst_nodes.reshape(ntok).astype(jnp.int32),
        adj_nodes.reshape(ntok).astype(jnp.int32),
        rel_positions.reshape(ntok).astype(jnp.int32),
        ctable.reshape(-1),
    )
    return out.reshape(bdim, sdim, D_MODEL)


# speedup vs baseline: 1.7072x; 1.4385x over previous
import functools

import jax
import jax.numpy as jnp
from jax import lax
from jax.experimental import pallas as pl
from jax.experimental.pallas import tpu as pltpu
from jax.experimental.pallas import tpu_sc as plsc

D_MODEL = 2048
D4 = D_MODEL // 4
N_COMBO = 44
NB = 4
CB = D_MODEL // NB

L = 16
K = 8
NBUF = 4


def _ctable_body(st_ref, at_ref, rt_ref, w_ref, b_ref, out_ref):
    w = w_ref[...]
    ps = jnp.dot(st_ref[...], w[0:D4, :], preferred_element_type=jnp.float32)
    pa = jnp.dot(at_ref[...], w[D4:2 * D4, :], preferred_element_type=jnp.float32)
    pr = jnp.dot(rt_ref[...], w[2 * D4:3 * D4, :], preferred_element_type=jnp.float32)
    bv = b_ref[...]
    for s in range(2):
        for a in range(2):
            out_ref[s * 2 + a, :, :] = pr + (ps[s] + pa[a] + bv)[None, :]


def _build_ctable(subst_table, adj_table, rel_table, W, b):
    c4 = pl.pallas_call(
        _ctable_body,
        grid=(NB,),
        in_specs=[
            pl.BlockSpec((2, D4), lambda c: (0, 0)),
            pl.BlockSpec((2, D4), lambda c: (0, 0)),
            pl.BlockSpec((11, D4), lambda c: (0, 0)),
            pl.BlockSpec((3 * D4, CB), lambda c: (0, c)),
            pl.BlockSpec((CB,), lambda c: (c,)),
        ],
        out_specs=pl.BlockSpec((4, 11, CB), lambda c: (0, 0, c)),
        out_shape=jax.ShapeDtypeStruct((4, 11, D_MODEL), jnp.float32),
    )(subst_table, adj_table, rel_table, W, b)
    return c4.reshape(N_COMBO, D_MODEL)


def _make_sc_lookup(ntok):
    info = plsc.get_sparse_core_info()
    nw = info.num_cores * info.num_subcores
    tpw = ntok // nw
    nch = tpw // K
    mesh = plsc.VectorSubcoreMesh(core_axis_name="c", subcore_axis_name="s")

    ngrp = tpw // K
    gelems = K * D_MODEL
    celems = N_COMBO * D_MODEL
    un = 16

    @functools.partial(
        pl.kernel,
        out_type=jax.ShapeDtypeStruct((ntok, D_MODEL), jnp.float32),
        mesh=mesh,
        compiler_params=pltpu.CompilerParams(needs_layout_passes=False),
        scratch_types=[
            pltpu.VMEM((tpw,), jnp.int32),
            pltpu.VMEM((tpw,), jnp.int32),
            pltpu.VMEM((tpw,), jnp.int32),
            pltpu.VMEM((celems,), jnp.float32),
            pltpu.VMEM((K, D_MODEL), jnp.float32),
            pltpu.VMEM((K, D_MODEL), jnp.float32),
            pltpu.SemaphoreType.DMA,
            pltpu.SemaphoreType.DMA,
            pltpu.SemaphoreType.DMA,
            pltpu.SemaphoreType.DMA,
        ],
    )
    def sc_lookup(sub_hbm, adj_hbm, rel_hbm, cf_hbm, c2d_hbm, out_hbm,
                  sv, av, rv, cloc, stag0, stag1, semc, semg, sw0, sw1):
        stag = (stag0, stag1)
        semw = (sw0, sw1)
        wid = lax.axis_index("s") * info.num_cores + lax.axis_index("c")
        base = wid * tpw
        pltpu.async_copy(cf_hbm, cloc, semc)
        pltpu.sync_copy(sub_hbm.at[pl.ds(base, tpw)], sv)
        pltpu.sync_copy(adj_hbm.at[pl.ds(base, tpw)], av)
        pltpu.sync_copy(rel_hbm.at[pl.ds(base, tpw)], rv)

        for i in range(tpw // L):
            s = sv[pl.ds(i * L, L)]
            a = av[pl.ds(i * L, L)]
            r = rv[pl.ds(i * L, L)]
            c = s * 22 + a * 11 + jnp.clip(r + 5, 0, 10)
            sv[pl.ds(i * L, L)] = jnp.clip(c, 0, N_COMBO - 1)

        pltpu.make_async_copy(cf_hbm, cloc, semc).wait()

        lane = lax.iota(jnp.int32, L)

        def wait_write(g, b):
            pltpu.make_async_copy(
                stag[b], out_hbm.at[pl.ds(base + g * K, K)],
                semw[b]).wait()

        def fire_write(g, b):
            pltpu.async_copy(
                stag[b], out_hbm.at[pl.ds(base + g * K, K)], semw[b])

        def fire_gather(g):
            pltpu.async_copy(
                c2d_hbm.at[sv.at[pl.ds(g * K, K)]], stag0, semg)

        def wait_gather(g):
            pltpu.make_async_copy(
                c2d_hbm.at[sv.at[pl.ds(g * K, K)]], stag0, semg).wait()

        def build_group(g, b):
            sb = stag[b]
            for r in range(K):
                perm = lax.bitwise_and(lane + r, K - 1)
                ci = plsc.load_gather(sv, [perm + g * K])
                src0 = ci * D_MODEL + lane

                def col_step(wi, carry, src0=src0, perm=perm):
                    offs = [(wi * un + j) * L for j in range(un)]
                    vals = [plsc.load_gather(cloc, [src0 + offs[j]])
                            for j in range(un)]
                    for j in range(un):
                        plsc.store_scatter(sb, [perm, lane + offs[j]],
                                           vals[j])
                    return carry

                lax.fori_loop(0, D_MODEL // (L * un), col_step, 0)

        def do_pair(p, first):
            g0 = 2 * p
            g1 = g0 + 1
            if not first:
                wait_write(g0 - 2, 0)
            fire_gather(g0)
            if not first:
                wait_write(g1 - 2, 1)
            build_group(g1, 1)
            wait_gather(g0)
            fire_write(g0, 0)
            fire_write(g1, 1)

        do_pair(0, True)
        lax.fori_loop(1, ngrp // 2,
                      lambda p, c: (do_pair(p, False), c)[1], 0)
        wait_write(ngrp - 2, 0)
        wait_write(ngrp - 1, 1)

    return sc_lookup


def kernel(subst_nodes, adj_nodes, rel_positions, subst_table, adj_table,
           rel_table, W, b):
    bdim, sdim = subst_nodes.shape
    ntok = bdim * sdim
    ctable = _build_ctable(subst_table, adj_table, rel_table, W, b)
    sc_lookup = _make_sc_lookup(ntok)
    out = sc_lookup(
        subst_nodes.reshape(ntok).astype(jnp.int32),
        adj_nodes.reshape(ntok).astype(jnp.int32),
        rel_positions.reshape(ntok).astype(jnp.int32),
        ctable.reshape(-1),
        ctable,
    )
    return out.reshape(bdim, sdim, D_MODEL)
